# baseline (speedup 1.0000x reference)
import functools
import math

import jax
import jax.numpy as jnp
from jax import lax
from jax.experimental import pallas as pl
from jax.experimental.pallas import tpu as pltpu
from jax.experimental.pallas import tpu_sc as plsc

D_MODEL = 64
SCALE = math.sqrt(D_MODEL)

_info = plsc.get_sparse_core_info()
_NC = _info.num_cores
_NS = _info.num_subcores
_L = _info.num_lanes
_NW = _NC * _NS

TOK = 128
NBUF = 4
NOUT = 4
PREFETCH = 2


@jax.jit
def kernel(x, lut):
    NB, T = x.shape
    V = lut.shape[0]
    B = NB * T
    n_blocks_total = B // TOK
    blocks_per_w = n_blocks_total // _NW
    tok_per_w = blocks_per_w * TOK
    bb_per_t = NB // TOK

    xt = x.T.reshape(B).astype(jnp.int32)
    lut2 = lut.reshape(V // 2, 2 * D_MODEL)

    mesh = plsc.VectorSubcoreMesh(core_axis_name="c", subcore_axis_name="s")

    @functools.partial(
        pl.kernel,
        mesh=mesh,
        out_type=jax.ShapeDtypeStruct((T * D_MODEL, NB), jnp.float32),
        scratch_types=[
            pltpu.VMEM((tok_per_w,), jnp.int32),
            [pltpu.VMEM((TOK, 2 * D_MODEL), jnp.float32) for _ in range(NBUF)],
            [pltpu.VMEM((TOK,), jnp.int32) for _ in range(NBUF)],
            [pltpu.VMEM((TOK,), jnp.int32) for _ in range(NBUF)],
            [pltpu.VMEM((D_MODEL, TOK), jnp.float32) for _ in range(NOUT)],
            [pltpu.SemaphoreType.DMA for _ in range(NBUF)],
            [pltpu.SemaphoreType.DMA for _ in range(NOUT)],
        ],
        compiler_params=pltpu.CompilerParams(
            use_tc_tiling_on_sc=True, needs_layout_passes=False
        ),
    )
    def gather_t(idx_hbm, table_hbm, out_hbm, idx_v, rows, rowids, colbase,
                 blocks, sg, ss):
        wid = lax.axis_index("s") * _NC + lax.axis_index("c")
        blk0 = wid * blocks_per_w

        pltpu.sync_copy(idx_hbm.at[pl.ds(blk0 * TOK, tok_per_w)], idx_v)

        def prep_and_gather(g, b):
            for gg in range(TOK // _L):
                sl = pl.ds(gg * _L, _L)
                v = idx_v[pl.ds(g * TOK + gg * _L, _L)]
                rowids[b][sl] = lax.shift_right_logical(v, 1)
                colbase[b][sl] = lax.shift_left(jnp.bitwise_and(v, 1), 6)
            pltpu.async_copy(table_hbm.at[rowids[b]], rows[b], sg[b])

        def out_slice(g):
            fb = blk0 + g
            t = lax.shift_right_logical(fb, 5)
            bb = jnp.bitwise_and(fb, bb_per_t - 1)
            return out_hbm.at[pl.ds(t * D_MODEL, D_MODEL),
                              pl.ds(bb * TOK, TOK)]

        def wait_store(g_prev, o):
            pltpu.make_async_copy(blocks[o], out_slice(g_prev), ss[o]).wait()

        def transpose_scale(b, o):
            for gg in range(TOK // _L):
                row_v = lax.iota(jnp.int32, _L) + gg * _L
                cb_v = colbase[b][pl.ds(gg * _L, _L)]

                def f_body(fi, c):
                    f0 = fi * 4
                    for df in range(4):
                        f = f0 + df
                        vals = plsc.load_gather(rows[b], [row_v, cb_v + f])
                        blocks[o][f, pl.ds(gg * _L, _L)] = vals * SCALE
                    return c

                lax.fori_loop(0, D_MODEL // 4, f_body, 0, unroll=False)

        def process(g, b, prefetch, first_round):
            pltpu.make_async_copy(table_hbm.at[rowids[b]], rows[b],
                                  sg[b]).wait()
            if first_round:
                transpose_scale(b, b)
            else:
                wait_store(g - NOUT, b)
                transpose_scale(b, b)
            pltpu.async_copy(blocks[b], out_slice(g), ss[b])
            if prefetch:
                prep_and_gather(g + PREFETCH, (b + PREFETCH) % NBUF)

        for g in range(PREFETCH):
            prep_and_gather(g, g % NBUF)

        for b in range(NBUF):
            process(b, b, prefetch=True, first_round=True)

        def loop_body(k, c):
            g0 = (k + 1) * NBUF
            for b in range(NBUF):
                process(g0 + b, b, prefetch=True, first_round=False)
            return c

        n_main = blocks_per_w // NBUF - 2
        lax.fori_loop(0, n_main, loop_body, 0, unroll=False)

        g0 = blocks_per_w - NBUF
        for b in range(NBUF):
            process(g0 + b, b, prefetch=(b < NBUF - PREFETCH),
                    first_round=False)
        for b in range(NBUF):
            wait_store(g0 + b, b)

    out2d = gather_t(xt, lut2)
    return out2d.reshape(T, D_MODEL, NB).transpose(2, 0, 1)


# speedup vs baseline: 1.5382x; 1.5382x over previous
import functools
import math

import jax
import jax.numpy as jnp
from jax import lax
from jax.experimental import pallas as pl
from jax.experimental.pallas import tpu as pltpu
from jax.experimental.pallas import tpu_sc as plsc

D_MODEL = 64
SCALE = math.sqrt(D_MODEL)

_info = plsc.get_sparse_core_info()
_NC = _info.num_cores
_NS = _info.num_subcores
_L = _info.num_lanes
_NW = _NC * _NS

TOK = 128
NBUF = 4
NOUT = 4
PREFETCH = 2


@jax.jit
def kernel(x, lut):
    NB, T = x.shape
    V = lut.shape[0]
    B = NB * T
    n_blocks_total = B // TOK
    blocks_per_w = n_blocks_total // _NW
    tok_per_w = blocks_per_w * TOK
    bb_per_t = NB // TOK

    xt = x.T.reshape(B).astype(jnp.int32)
    lut2 = lut.reshape(V // 2, 2 * D_MODEL)

    mesh = plsc.VectorSubcoreMesh(core_axis_name="c", subcore_axis_name="s")

    @functools.partial(
        pl.kernel,
        mesh=mesh,
        out_type=jax.ShapeDtypeStruct((T * D_MODEL, NB), jnp.float32),
        scratch_types=[
            pltpu.VMEM((tok_per_w,), jnp.int32),
            [pltpu.VMEM((TOK, 2 * D_MODEL), jnp.float32) for _ in range(NBUF)],
            [pltpu.VMEM((TOK,), jnp.int32) for _ in range(NBUF)],
            [pltpu.VMEM((TOK,), jnp.int32) for _ in range(NBUF)],
            [pltpu.VMEM((D_MODEL, TOK), jnp.float32) for _ in range(NOUT)],
            [pltpu.SemaphoreType.DMA for _ in range(NBUF)],
            [pltpu.SemaphoreType.DMA for _ in range(NOUT)],
        ],
        compiler_params=pltpu.CompilerParams(
            use_tc_tiling_on_sc=True, needs_layout_passes=False
        ),
    )
    def gather_t(idx_hbm, table_hbm, out_hbm, idx_v, rows, rowids, colbase,
                 blocks, sg, ss):
        wid = lax.axis_index("s") * _NC + lax.axis_index("c")
        blk0 = wid * blocks_per_w

        pltpu.sync_copy(idx_hbm.at[pl.ds(blk0 * TOK, tok_per_w)], idx_v)

        def prep_and_gather(g, b):
            for gg in range(TOK // _L):
                sl = pl.ds(gg * _L, _L)
                v = idx_v[pl.ds(g * TOK + gg * _L, _L)]
                rowids[b][sl] = lax.shift_right_logical(v, 1)
                colbase[b][sl] = lax.shift_left(jnp.bitwise_and(v, 1), 6)
            pltpu.async_copy(table_hbm.at[rowids[b]], rows[b], sg[b])

        def out_slice(g):
            fb = blk0 + g
            t = lax.shift_right_logical(fb, 5)
            bb = jnp.bitwise_and(fb, bb_per_t - 1)
            return out_hbm.at[pl.ds(t * D_MODEL, D_MODEL),
                              pl.ds(bb * TOK, TOK)]

        def wait_store(g_prev, o):
            pltpu.make_async_copy(blocks[o], out_slice(g_prev), ss[o]).wait()

        def transpose_scale(b, o):
            for gg in range(TOK // _L):
                row_v = lax.iota(jnp.int32, _L) + gg * _L
                cb_v = colbase[b][pl.ds(gg * _L, _L)]
                sl = pl.ds(gg * _L, _L)

                @plsc.parallel_loop(0, D_MODEL, step=1, unroll=8)
                def f_body(f):
                    vals = plsc.load_gather(rows[b], [row_v, cb_v + f])
                    blocks[o][f, sl] = vals * SCALE

        def process(g, b, prefetch, first_round):
            pltpu.make_async_copy(table_hbm.at[rowids[b]], rows[b],
                                  sg[b]).wait()
            if first_round:
                transpose_scale(b, b)
            else:
                wait_store(g - NOUT, b)
                transpose_scale(b, b)
            pltpu.async_copy(blocks[b], out_slice(g), ss[b])
            if prefetch:
                prep_and_gather(g + PREFETCH, (b + PREFETCH) % NBUF)

        for g in range(PREFETCH):
            prep_and_gather(g, g % NBUF)

        for b in range(NBUF):
            process(b, b, prefetch=True, first_round=True)

        def loop_body(k, c):
            g0 = (k + 1) * NBUF
            for b in range(NBUF):
                process(g0 + b, b, prefetch=True, first_round=False)
            return c

        n_main = blocks_per_w // NBUF - 2
        lax.fori_loop(0, n_main, loop_body, 0, unroll=False)

        g0 = blocks_per_w - NBUF
        for b in range(NBUF):
            process(g0 + b, b, prefetch=(b < NBUF - PREFETCH),
                    first_round=False)
        for b in range(NBUF):
            wait_store(g0 + b, b)

    out2d = gather_t(xt, lut2)
    return out2d.reshape(T, D_MODEL, NB).transpose(2, 0, 1)
